# baseline (speedup 1.0000x reference)
import functools

import jax
import jax.numpy as jnp
from jax import lax
from jax.experimental import pallas as pl
from jax.experimental.pallas import tpu as pltpu
from jax.experimental.pallas import tpu_sc as plsc

NC = 2
NS = 16
NW = NC * NS

C = 8
NBUF = 8
GLOOK = 4


def _embedding_call(wte, ids2d):
    Brows, S = ids2d.shape
    V, D = wte.shape
    B_total = Brows * S
    b_per_w = B_total // NW
    NCHUNK = b_per_w // C
    NSTEP = NCHUNK // NBUF
    w_per_row = S // b_per_w

    mesh = plsc.VectorSubcoreMesh(
        core_axis_name="c", subcore_axis_name="s", num_cores=NC, num_subcores=NS
    )

    @functools.partial(
        pl.kernel,
        out_type=jax.ShapeDtypeStruct((B_total, D), jnp.float32),
        mesh=mesh,
        scratch_types=[
            pltpu.VMEM((b_per_w,), jnp.int32),
            pltpu.VMEM((NBUF, C, D), jnp.float32),
        ]
        + [pltpu.SemaphoreType.DMA] * (2 * NBUF),
    )
    def body(wte_h, idx_h, out_h, idx_v, bufs, *sems):
        s_in = sems[:NBUF]
        s_out = sems[NBUF:]
        cid = lax.axis_index("c")
        sid = lax.axis_index("s")
        wid = sid * NC + cid
        base = wid * b_per_w
        row = wid // w_per_row
        col0 = (wid % w_per_row) * b_per_w

        pltpu.sync_copy(idx_h.at[row, pl.ds(col0, b_per_w)], idx_v)

        def gather_cp(g, b):
            off = pl.multiple_of(g * C, C)
            return pltpu.make_async_copy(
                wte_h.at[idx_v.at[pl.ds(off, C)]], bufs.at[b], s_in[b]
            )

        def out_cp(g, b):
            return pltpu.make_async_copy(
                bufs.at[b], out_h.at[pl.ds(base + g * C, C)], s_out[b]
            )

        for g in range(GLOOK):
            gather_cp(g, g).start()

        def slot(g, j, wait_out):
            b = j
            gather_cp(g, b).wait()
            out_cp(g, b).start()
            ng = g + GLOOK
            bb = (j + GLOOK) % NBUF
            if wait_out:
                out_cp(g, bb).wait()

                @pl.when(ng < NCHUNK)
                def _():
                    gather_cp(ng, bb).start()
            else:
                gather_cp(ng, bb).start()

        for j in range(NBUF):
            slot(j, j, wait_out=(j + GLOOK >= NBUF))

        @pl.loop(1, NSTEP)
        def _(step):
            g0 = step * NBUF
            for j in range(NBUF):
                slot(g0 + j, j, wait_out=True)

        for g in range(NCHUNK - (NBUF - GLOOK), NCHUNK):
            out_cp(g, g % NBUF).wait()

    return body(wte, ids2d)


def kernel(input_ids, wte):
    in_shape = input_ids.shape
    D = wte.shape[1]
    ids2d = input_ids.reshape(-1, in_shape[-1]).astype(jnp.int32)
    out = _embedding_call(wte, ids2d)
    return out.reshape(in_shape[0], in_shape[-1], D)


# speedup vs baseline: 1.0109x; 1.0109x over previous
import functools

import jax
import jax.numpy as jnp
from jax import lax
from jax.experimental import pallas as pl
from jax.experimental.pallas import tpu as pltpu
from jax.experimental.pallas import tpu_sc as plsc

NC = 2
NS = 16
NW = NC * NS

C = 16
NBUF = 4


def _embedding_call(wte, ids2d):
    Brows, S = ids2d.shape
    V, D = wte.shape
    B_total = Brows * S
    b_per_w = B_total // NW
    NCHUNK = b_per_w // C
    NSTEP = NCHUNK // NBUF
    w_per_row = S // b_per_w

    mesh = plsc.VectorSubcoreMesh(
        core_axis_name="c", subcore_axis_name="s", num_cores=NC, num_subcores=NS
    )

    @functools.partial(
        pl.kernel,
        out_type=jax.ShapeDtypeStruct((B_total, D), jnp.float32),
        mesh=mesh,
        scratch_types=[
            pltpu.VMEM((b_per_w,), jnp.int32),
            pltpu.VMEM((NBUF, C, D), jnp.float32),
        ]
        + [pltpu.SemaphoreType.DMA] * (2 * NBUF),
    )
    def body(wte_h, idx_h, out_h, idx_v, bufs, *sems):
        s_in = sems[:NBUF]
        s_out = sems[NBUF:]
        cid = lax.axis_index("c")
        sid = lax.axis_index("s")
        wid = sid * NC + cid
        base = wid * b_per_w
        row = wid // w_per_row
        col0 = (wid % w_per_row) * b_per_w

        pltpu.sync_copy(idx_h.at[row, pl.ds(col0, b_per_w)], idx_v)

        def gather(g, b):
            off = pl.multiple_of(g * C, C)
            pltpu.make_async_copy(
                wte_h.at[idx_v.at[pl.ds(off, C)]], bufs.at[b], s_in[b]
            ).start()

        for b in range(NBUF):
            gather(b, b)

        @pl.loop(0, NSTEP)
        def _(step):
            g0 = step * NBUF
            for b in range(NBUF):
                g = g0 + b
                pltpu.make_async_copy(
                    wte_h.at[idx_v.at[pl.ds(pl.multiple_of(g * C, C), C)]],
                    bufs.at[b],
                    s_in[b],
                ).wait()
                out_cp = pltpu.make_async_copy(
                    bufs.at[b],
                    out_h.at[pl.ds(base + g * C, C)],
                    s_out[b],
                )
                out_cp.start()
                out_cp.wait()

                @pl.when(g < NCHUNK - NBUF)
                def _():
                    gather(g + NBUF, b)

    return body(wte, ids2d)


def kernel(input_ids, wte):
    in_shape = input_ids.shape
    D = wte.shape[1]
    ids2d = input_ids.reshape(-1, in_shape[-1]).astype(jnp.int32)
    out = _embedding_call(wte, ids2d)
    return out.reshape(in_shape[0], in_shape[-1], D)
